# baseline (speedup 1.0000x reference)
import jax
import jax.numpy as jnp
from jax import lax
from jax.experimental import pallas as pl
from jax.experimental.pallas import tpu as pltpu
from jax.experimental.pallas import tpu_sc as plsc

B, F, P = 1024, 26, 40
VOCAB, EMB = 100000, 32

N = B * F * P
NC, NS = 2, 16
NW = NC * NS
BW = B // NW
FP = F * P
HB = BW // 2
SR = HB * P
NS_T = 2 * F
NBUF = 4


def _sc_gather(table, idxfp):
    mesh = plsc.VectorSubcoreMesh(core_axis_name="c", subcore_axis_name="s")

    @pl.kernel(
        out_type=jax.ShapeDtypeStruct((N, EMB), jnp.float32),
        mesh=mesh,
        scratch_types=[
            pltpu.VMEM((FP, BW), jnp.int32),
            [pltpu.VMEM((SR,), jnp.int32) for _ in range(NBUF)],
            [pltpu.VMEM((SR, EMB), jnp.float32) for _ in range(NBUF)],
            pltpu.SemaphoreType.DMA,
            pltpu.SemaphoreType.DMA,
            [pltpu.SemaphoreType.DMA for _ in range(NBUF)],
            [pltpu.SemaphoreType.DMA for _ in range(NBUF)],
        ],
        compiler_params=pltpu.CompilerParams(use_tc_tiling_on_sc=False,
                                             needs_layout_passes=False),
    )
    def k(table_hbm, idx_hbm, out_hbm, idx_v, idx_t, rows, isemA, isemB,
          gsem, ssem):
        wid = lax.axis_index("s") * NC + lax.axis_index("c")
        b0 = wid * BW

        HEAD = 2 * P
        cA = pltpu.make_async_copy(
            idx_hbm.at[pl.ds(0, HEAD), pl.ds(b0, BW)],
            idx_v.at[pl.ds(0, HEAD)], isemA)
        cB = pltpu.make_async_copy(
            idx_hbm.at[pl.ds(HEAD, FP - HEAD), pl.ds(b0, BW)],
            idx_v.at[pl.ds(HEAD, FP - HEAD)], isemB)
        cA.start()
        cB.start()
        cA.wait()

        iota = lax.iota(jnp.int32, 16)

        def permute(s, b):
            base = (s // 2) * P
            boff = (s % 2) * HB

            def vec(j, _):
                kv = iota + j * 16
                pv = lax.rem(kv, P)
                bv = lax.div(kv, P)
                g = plsc.load_gather(idx_v, [pv + base, bv + boff])
                idx_t[b][pl.ds(j * 16, 16)] = g
                return _
            lax.fori_loop(0, SR // 16, vec, None)

        def gather(b):
            return pltpu.make_async_copy(table_hbm.at[idx_t[b]], rows[b],
                                         gsem[b])

        def store(s, b):
            f = s // 2
            return pltpu.make_async_copy(
                rows[b],
                out_hbm.at[pl.ds((f * B + b0) * P + (s % 2) * SR, SR)],
                ssem[b])

        def body(u, _):
            @pl.when(u == 1)
            def _wb():
                cB.wait()

            for b in range(NBUF):
                s = u * NBUF + b

                @pl.when(u > 0)
                def _drain():
                    store(s, b).wait()

                permute(s, b)
                gather(b).start()
            for b in range(NBUF):
                s = u * NBUF + b
                gather(b).wait()
                store(s, b).start()
            return _

        lax.fori_loop(0, NS_T // NBUF, body, None)
        for b in range(NBUF):
            store(b, b).wait()

    return k(table, idxfp)


def kernel(feature, table):
    idxfp = feature.transpose(1, 2, 0).reshape(FP, B)
    out = _sc_gather(table, idxfp)
    return out.reshape(F, B, P * EMB).transpose(1, 0, 2)


# speedup vs baseline: 1.0060x; 1.0060x over previous
import jax
import jax.numpy as jnp
from jax import lax
from jax.experimental import pallas as pl
from jax.experimental.pallas import tpu as pltpu
from jax.experimental.pallas import tpu_sc as plsc

B, F, P = 1024, 26, 40
VOCAB, EMB = 100000, 32

N = B * F * P
NC, NS = 2, 16
NW = NC * NS
BW = B // NW
FP = F * P
HB = BW // 2
SR = HB * P
NS_T = 2 * F
NBUF = 4


def _sc_gather(table, idxfp):
    mesh = plsc.VectorSubcoreMesh(core_axis_name="c", subcore_axis_name="s")

    @pl.kernel(
        out_type=jax.ShapeDtypeStruct((N, EMB), jnp.float32),
        mesh=mesh,
        scratch_types=[
            pltpu.VMEM((FP, BW), jnp.int32),
            [pltpu.VMEM((SR,), jnp.int32) for _ in range(NBUF)],
            [pltpu.VMEM((SR, EMB), jnp.float32) for _ in range(NBUF)],
            pltpu.SemaphoreType.DMA,
            pltpu.SemaphoreType.DMA,
            [pltpu.SemaphoreType.DMA for _ in range(NBUF)],
            [pltpu.SemaphoreType.DMA for _ in range(NBUF)],
        ],
        compiler_params=pltpu.CompilerParams(use_tc_tiling_on_sc=False,
                                             needs_layout_passes=False),
    )
    def k(table_hbm, idx_hbm, out_hbm, idx_v, idx_t, rows, isemA, isemB,
          gsem, ssem):
        wid = lax.axis_index("s") * NC + lax.axis_index("c")
        b0 = wid * BW

        HEAD = 2 * P
        cA = pltpu.make_async_copy(
            idx_hbm.at[pl.ds(0, HEAD), b0 // 128, pl.ds(b0 % 128, BW)],
            idx_v.at[pl.ds(0, HEAD)], isemA)
        cB = pltpu.make_async_copy(
            idx_hbm.at[pl.ds(HEAD, FP - HEAD), b0 // 128,
                       pl.ds(b0 % 128, BW)],
            idx_v.at[pl.ds(HEAD, FP - HEAD)], isemB)
        cA.start()
        cB.start()
        cA.wait()

        iota = lax.iota(jnp.int32, 16)

        def permute(s, b):
            base = (s // 2) * P
            boff = (s % 2) * HB

            def vec(j, _):
                kv = iota + j * 16
                pv = lax.rem(kv, P)
                bv = lax.div(kv, P)
                g = plsc.load_gather(idx_v, [pv + base, bv + boff])
                idx_t[b][pl.ds(j * 16, 16)] = g
                return _
            lax.fori_loop(0, SR // 16, vec, None)

        def gather(b):
            return pltpu.make_async_copy(table_hbm.at[idx_t[b]], rows[b],
                                         gsem[b])

        def store(s, b):
            f = s // 2
            return pltpu.make_async_copy(
                rows[b],
                out_hbm.at[pl.ds((f * B + b0) * P + (s % 2) * SR, SR)],
                ssem[b])

        def body(u, _):
            @pl.when(u == 1)
            def _wb():
                cB.wait()

            for b in range(NBUF):
                s = u * NBUF + b

                @pl.when(u > 0)
                def _drain():
                    store(s, b).wait()

                permute(s, b)
                gather(b).start()
            for b in range(NBUF):
                s = u * NBUF + b
                gather(b).wait()
                store(s, b).start()
            return _

        lax.fori_loop(0, NS_T // NBUF, body, None)
        for b in range(NBUF):
            store(b, b).wait()

    return k(table, idxfp)


def kernel(feature, table):
    idxfp = feature.transpose(1, 2, 0).reshape(FP, B // 128, 128)
    out = _sc_gather(table, idxfp)
    return out.reshape(F, B, P * EMB).transpose(1, 0, 2)
